# baseline (speedup 1.0000x reference)
import functools

import jax
import jax.numpy as jnp
from jax import lax
from jax.experimental import pallas as pl
from jax.experimental.pallas import tpu as pltpu
from jax.experimental.pallas import tpu_sc as plsc

N = 10000
E = 320000
D_IN = 128
H = 128
N_HOPS = 6

NC = 2
NS = 16
NW = NC * NS
BATCH = 128
NBW = 80
CHUNK = 16
NBA = 160
NBB = 2 * NBW - NBA
TOTB = NW * NBW
E_PAD = TOTB * BATCH
N_PAD = 10240
RP = N_PAD // NS
RC = RP // BATCH


def _mesh():
    return plsc.VectorSubcoreMesh(
        core_axis_name="c", subcore_axis_name="s",
        num_cores=NC, num_subcores=NS)


def _sc_degree(dst2):
    @functools.partial(
        pl.kernel,
        out_type=(jax.ShapeDtypeStruct((N_PAD,), jnp.float32),
                  jax.ShapeDtypeStruct((N_PAD,), jnp.float32)),
        mesh=_mesh(),
        scratch_types=[
            pltpu.VMEM((NBW, BATCH), jnp.int32),
            pltpu.VMEM((BATCH,), jnp.float32),
            pltpu.VMEM((BATCH,), jnp.float32),
            pltpu.VMEM_SHARED((N_PAD,), jnp.float32),
        ],
    )
    def deg_kernel(dst_hbm, d0_out, d1_out, dst_v, ones_v, zb_v, deg_sh):
        c = lax.axis_index("c")
        s = lax.axis_index("s")
        w = s * NC + c
        for i in range(BATCH // 16):
            ones_v[pl.ds(i * 16, 16)] = jnp.ones((16,), jnp.float32)
            zb_v[pl.ds(i * 16, 16)] = jnp.zeros((16,), jnp.float32)
        pltpu.sync_copy(dst_hbm.at[pl.ds(w * NBW, NBW)], dst_v)
        base = s * RP
        for t in range(RC):
            pltpu.sync_copy(zb_v, deg_sh.at[pl.ds(base + t * BATCH, BATCH)])
        plsc.subcore_barrier()

        def body(j, carry):
            pltpu.sync_copy(ones_v, deg_sh.at[dst_v.at[j]], add=True)
            return carry

        lax.fori_loop(0, NBW, body, 0)
        plsc.subcore_barrier()

        @pl.when(c == 0)
        def _():
            for t in range(RC):
                off = base + t * BATCH
                pltpu.sync_copy(deg_sh.at[pl.ds(off, BATCH)],
                                d0_out.at[pl.ds(off, BATCH)])

        @pl.when(c == 1)
        def _():
            for t in range(RC):
                off = base + t * BATCH
                pltpu.sync_copy(deg_sh.at[pl.ds(off, BATCH)],
                                d1_out.at[pl.ds(off, BATCH)])

    return deg_kernel(dst2)


def _sc_hop(table, src2, dst2):
    @functools.partial(
        pl.kernel,
        out_type=(jax.ShapeDtypeStruct((N_PAD, H), jnp.float32),
                  jax.ShapeDtypeStruct((N_PAD, H), jnp.float32)),
        mesh=_mesh(),
        scratch_types=[
            pltpu.VMEM((CHUNK, BATCH), jnp.int32),
            pltpu.VMEM((CHUNK, BATCH), jnp.int32),
            pltpu.VMEM((2, BATCH, H), jnp.float32),
            pltpu.SemaphoreType.DMA,
            pltpu.SemaphoreType.DMA,
            pltpu.VMEM_SHARED((N_PAD, H), jnp.float32),
        ],
    )
    def hop_kernel(table_hbm, src_hbm, dst_hbm, p0_out, p1_out,
                   src_v, dst_v, rows_v, gsem0, gsem1, agg_sh):
        c = lax.axis_index("c")
        s = lax.axis_index("s")
        base_b = jnp.where(c == 0, s * NBA, NS * NBA + s * NBB)
        nchunks = jnp.where(c == 0, NBA // CHUNK, NBB // CHUNK)

        def zrow(i, carry):
            for cc in range(H // 16):
                rows_v[0, i, pl.ds(cc * 16, 16)] = jnp.zeros(
                    (16,), jnp.float32)
            return carry

        lax.fori_loop(0, BATCH, zrow, 0)
        base = s * RP
        for t in range(RC):
            pltpu.sync_copy(rows_v.at[0],
                            agg_sh.at[pl.ds(base + t * BATCH, BATCH)])
        plsc.subcore_barrier()

        def chunk_body(g, carry):
            off = base_b + g * CHUNK
            pltpu.sync_copy(src_hbm.at[pl.ds(off, CHUNK)], src_v)
            pltpu.sync_copy(dst_hbm.at[pl.ds(off, CHUNK)], dst_v)
            pltpu.async_copy(table_hbm.at[src_v.at[0]], rows_v.at[0], gsem0)
            pltpu.async_copy(table_hbm.at[src_v.at[1]], rows_v.at[1], gsem1)

            def pair_body(q, carry2):
                for p, sem in ((0, gsem0), (1, gsem1)):
                    j = 2 * q + p
                    pltpu.make_async_copy(
                        table_hbm.at[src_v.at[j]], rows_v.at[p], sem).wait()
                    pltpu.sync_copy(rows_v.at[p], agg_sh.at[dst_v.at[j]],
                                    add=True)

                    @pl.when(q < CHUNK // 2 - 1)
                    def _():
                        pltpu.async_copy(
                            table_hbm.at[src_v.at[j + 2]], rows_v.at[p], sem)
                return carry2

            lax.fori_loop(0, CHUNK // 2, pair_body, 0)
            return carry

        lax.fori_loop(0, nchunks, chunk_body, 0)
        plsc.subcore_barrier()

        @pl.when(c == 0)
        def _():
            for t in range(RC):
                off = base + t * BATCH
                pltpu.sync_copy(agg_sh.at[pl.ds(off, BATCH)],
                                p0_out.at[pl.ds(off, BATCH)])

        @pl.when(c == 1)
        def _():
            for t in range(RC):
                off = base + t * BATCH
                pltpu.sync_copy(agg_sh.at[pl.ds(off, BATCH)],
                                p1_out.at[pl.ds(off, BATCH)])

    return hop_kernel(table, src2, dst2)


def _tc_combine(p0, p1, n2):
    RB = 1024
    grid = (N_PAD // RB,)

    def body(p0_ref, p1_ref, n2_ref, t_ref):
        t_ref[...] = (p0_ref[...] + p1_ref[...]) * n2_ref[...][:, None]

    return pl.pallas_call(
        body,
        grid=grid,
        in_specs=[
            pl.BlockSpec((RB, H), lambda i: (i, 0)),
            pl.BlockSpec((RB, H), lambda i: (i, 0)),
            pl.BlockSpec((RB,), lambda i: (i,)),
        ],
        out_specs=pl.BlockSpec((RB, H), lambda i: (i, 0)),
        out_shape=jax.ShapeDtypeStruct((N_PAD, H), jnp.float32),
    )(p0, p1, n2)


def _tc_prep(x_pad, W_in, b_in2, d0, d1):
    RB = 1024
    grid = (N_PAD // RB,)

    def body(x_ref, w_ref, b_ref, d0_ref, d1_ref, hs_ref, n2_ref, invn_ref):
        h = jnp.dot(x_ref[...], w_ref[...],
                    preferred_element_type=jnp.float32) + b_ref[...]
        h = jnp.maximum(h, 0.0)
        dg = jnp.maximum(d0_ref[...] + d1_ref[...], 1.0)
        norm = lax.rsqrt(dg)
        hs_ref[...] = h * norm[:, None]
        n2_ref[...] = 1.0 / dg
        invn_ref[...] = jnp.sqrt(dg)

    return pl.pallas_call(
        body,
        grid=grid,
        in_specs=[
            pl.BlockSpec((RB, D_IN), lambda i: (i, 0)),
            pl.BlockSpec((D_IN, H), lambda i: (0, 0)),
            pl.BlockSpec((1, H), lambda i: (0, 0)),
            pl.BlockSpec((RB,), lambda i: (i,)),
            pl.BlockSpec((RB,), lambda i: (i,)),
        ],
        out_specs=[
            pl.BlockSpec((RB, H), lambda i: (i, 0)),
            pl.BlockSpec((RB,), lambda i: (i,)),
            pl.BlockSpec((RB,), lambda i: (i,)),
        ],
        out_shape=[
            jax.ShapeDtypeStruct((N_PAD, H), jnp.float32),
            jax.ShapeDtypeStruct((N_PAD,), jnp.float32),
            jax.ShapeDtypeStruct((N_PAD,), jnp.float32),
        ],
    )(x_pad, W_in, b_in2, d0, d1)


def _tc_proj(ts, invn, W6, b_rn2, gamma2, beta2):
    RB = 1024
    grid = (N_PAD // RB,)
    nt = len(ts)

    def body(*refs):
        t_refs = refs[:nt]
        invn_ref, w_ref, b_ref, g_ref, be_ref, z_ref = refs[nt:]
        invn = invn_ref[...][:, None]
        acc = jnp.zeros((RB, H), jnp.float32)
        for k in range(N_HOPS):
            hk = t_refs[k][...] * invn
            acc = acc + jnp.dot(hk, w_ref[k],
                                preferred_element_type=jnp.float32)
        z = acc + b_ref[...]
        mu = jnp.mean(z, axis=-1, keepdims=True)
        zc = z - mu
        var = jnp.mean(zc * zc, axis=-1, keepdims=True)
        z_ref[...] = zc * lax.rsqrt(var + 1e-5) * g_ref[...] + be_ref[...]

    in_specs = [pl.BlockSpec((RB, H), lambda i: (i, 0))] * nt + [
        pl.BlockSpec((RB,), lambda i: (i,)),
        pl.BlockSpec((N_HOPS, H, H), lambda i: (0, 0, 0)),
        pl.BlockSpec((1, H), lambda i: (0, 0)),
        pl.BlockSpec((1, H), lambda i: (0, 0)),
        pl.BlockSpec((1, H), lambda i: (0, 0)),
    ]
    return pl.pallas_call(
        body,
        grid=grid,
        in_specs=in_specs,
        out_specs=pl.BlockSpec((RB, H), lambda i: (i, 0)),
        out_shape=jax.ShapeDtypeStruct((N_PAD, H), jnp.float32),
    )(*ts, invn, W6, b_rn2, gamma2, beta2)


def kernel(x, edge_index, W_in, b_in, W_rn, b_rn, gamma, beta):
    pad = E_PAD - E
    src2 = jnp.concatenate(
        [edge_index[0], jnp.zeros((pad,), jnp.int32)]).reshape(TOTB, BATCH)
    pad_dst = N + (jnp.arange(pad, dtype=jnp.int32) % (N_PAD - N))
    dst2 = jnp.concatenate([edge_index[1], pad_dst]).reshape(TOTB, BATCH)
    x_pad = jnp.pad(x, ((0, N_PAD - N), (0, 0)))

    d0, d1 = _sc_degree(dst2)
    hs0, n2, invn = _tc_prep(x_pad, W_in, b_in.reshape(1, H), d0, d1)
    ts = []
    table = hs0
    for _ in range(N_HOPS):
        p0, p1 = _sc_hop(table, src2, dst2)
        table = _tc_combine(p0, p1, n2)
        ts.append(table)
    z = _tc_proj(ts, invn, W_rn.reshape(N_HOPS, H, H),
                 b_rn.reshape(1, H), gamma.reshape(1, H), beta.reshape(1, H))
    return z[:N]


# speedup vs baseline: 4.1180x; 4.1180x over previous
import functools

import jax
import jax.numpy as jnp
from jax import lax
from jax.experimental import pallas as pl
from jax.experimental.pallas import tpu as pltpu
from jax.experimental.pallas import tpu_sc as plsc

N = 10000
E = 320000
D_IN = 128
H = 128
N_HOPS = 6

NC = 2
NS = 16
NW = NC * NS
BATCH = 128
NBW = 80
CHUNK = 16
NBA = 80
NBB = 2 * NBW - NBA
TOTB = NW * NBW
E_PAD = TOTB * BATCH
N_PAD = 10240
RP = N_PAD // NS
RC = RP // BATCH


def _mesh():
    return plsc.VectorSubcoreMesh(
        core_axis_name="c", subcore_axis_name="s",
        num_cores=NC, num_subcores=NS)


def _sc_degree(dst2):
    @functools.partial(
        pl.kernel,
        out_type=(jax.ShapeDtypeStruct((N_PAD,), jnp.float32),
                  jax.ShapeDtypeStruct((N_PAD,), jnp.float32)),
        mesh=_mesh(),
        scratch_types=[
            pltpu.VMEM((NBW, BATCH), jnp.int32),
            pltpu.VMEM((BATCH,), jnp.float32),
            pltpu.VMEM((BATCH,), jnp.float32),
            pltpu.VMEM_SHARED((N_PAD,), jnp.float32),
        ],
    )
    def deg_kernel(dst_hbm, d0_out, d1_out, dst_v, ones_v, zb_v, deg_sh):
        c = lax.axis_index("c")
        s = lax.axis_index("s")
        w = s * NC + c
        for i in range(BATCH // 16):
            ones_v[pl.ds(i * 16, 16)] = jnp.ones((16,), jnp.float32)
            zb_v[pl.ds(i * 16, 16)] = jnp.zeros((16,), jnp.float32)
        pltpu.sync_copy(dst_hbm.at[pl.ds(w * NBW, NBW)], dst_v)
        base = s * RP
        for t in range(RC):
            pltpu.sync_copy(zb_v, deg_sh.at[pl.ds(base + t * BATCH, BATCH)])
        plsc.subcore_barrier()

        def body(j, carry):
            pltpu.sync_copy(ones_v, deg_sh.at[dst_v.at[j]], add=True)
            return carry

        lax.fori_loop(0, NBW, body, 0)
        plsc.subcore_barrier()

        @pl.when(c == 0)
        def _():
            for t in range(RC):
                off = base + t * BATCH
                pltpu.sync_copy(deg_sh.at[pl.ds(off, BATCH)],
                                d0_out.at[pl.ds(off, BATCH)])

        @pl.when(c == 1)
        def _():
            for t in range(RC):
                off = base + t * BATCH
                pltpu.sync_copy(deg_sh.at[pl.ds(off, BATCH)],
                                d1_out.at[pl.ds(off, BATCH)])

    return deg_kernel(dst2)


def _sc_hop(table, src2, dst2):
    @functools.partial(
        pl.kernel,
        out_type=(jax.ShapeDtypeStruct((N_PAD, H), jnp.float32),
                  jax.ShapeDtypeStruct((N_PAD, H), jnp.float32)),
        mesh=_mesh(),
        scratch_types=[
            pltpu.VMEM((CHUNK, BATCH), jnp.int32),
            pltpu.VMEM((CHUNK, BATCH), jnp.int32),
            pltpu.VMEM((2, BATCH, H), jnp.float32),
            pltpu.SemaphoreType.DMA,
            pltpu.SemaphoreType.DMA,
            pltpu.VMEM_SHARED((N_PAD, H), jnp.float32),
        ],
    )
    def hop_kernel(table_hbm, src_hbm, dst_hbm, p0_out, p1_out,
                   src_v, dst_v, rows_v, gsem0, gsem1, agg_sh):
        c = lax.axis_index("c")
        s = lax.axis_index("s")
        base_b = jnp.where(c == 0, s * NBA, NS * NBA + s * NBB)
        nchunks = jnp.where(c == 0, NBA // CHUNK, NBB // CHUNK)

        def zrow(i, carry):
            for cc in range(H // 16):
                rows_v[0, i, pl.ds(cc * 16, 16)] = jnp.zeros(
                    (16,), jnp.float32)
            return carry

        lax.fori_loop(0, BATCH, zrow, 0)
        base = s * RP
        for t in range(RC):
            pltpu.sync_copy(rows_v.at[0],
                            agg_sh.at[pl.ds(base + t * BATCH, BATCH)])
        plsc.subcore_barrier()

        def chunk_body(g, carry):
            off = base_b + g * CHUNK
            pltpu.sync_copy(src_hbm.at[pl.ds(off, CHUNK)], src_v)
            pltpu.sync_copy(dst_hbm.at[pl.ds(off, CHUNK)], dst_v)
            pltpu.async_copy(table_hbm.at[src_v.at[0]], rows_v.at[0], gsem0)
            pltpu.async_copy(table_hbm.at[src_v.at[1]], rows_v.at[1], gsem1)

            def pair_body(q, carry2):
                for p, sem in ((0, gsem0), (1, gsem1)):
                    j = 2 * q + p
                    pltpu.make_async_copy(
                        table_hbm.at[src_v.at[j]], rows_v.at[p], sem).wait()
                    pltpu.sync_copy(rows_v.at[p], agg_sh.at[dst_v.at[j]],
                                    add=True)

                    @pl.when(q < CHUNK // 2 - 1)
                    def _():
                        pltpu.async_copy(
                            table_hbm.at[src_v.at[j + 2]], rows_v.at[p], sem)
                return carry2

            lax.fori_loop(0, CHUNK // 2, pair_body, 0)
            return carry

        lax.fori_loop(0, nchunks, chunk_body, 0)
        plsc.subcore_barrier()

        @pl.when(c == 0)
        def _():
            for t in range(RC):
                off = base + t * BATCH
                pltpu.sync_copy(agg_sh.at[pl.ds(off, BATCH)],
                                p0_out.at[pl.ds(off, BATCH)])

        @pl.when(c == 1)
        def _():
            for t in range(RC):
                off = base + t * BATCH
                pltpu.sync_copy(agg_sh.at[pl.ds(off, BATCH)],
                                p1_out.at[pl.ds(off, BATCH)])

    return hop_kernel(table, src2, dst2)


def _tc_combine(p0, p1, n2):
    RB = 1024
    grid = (N_PAD // RB,)

    def body(p0_ref, p1_ref, n2_ref, t_ref):
        t_ref[...] = (p0_ref[...] + p1_ref[...]) * n2_ref[...][:, None]

    return pl.pallas_call(
        body,
        grid=grid,
        in_specs=[
            pl.BlockSpec((RB, H), lambda i: (i, 0)),
            pl.BlockSpec((RB, H), lambda i: (i, 0)),
            pl.BlockSpec((RB,), lambda i: (i,)),
        ],
        out_specs=pl.BlockSpec((RB, H), lambda i: (i, 0)),
        out_shape=jax.ShapeDtypeStruct((N_PAD, H), jnp.float32),
    )(p0, p1, n2)


def _tc_prep(x_pad, W_in, b_in2, d0, d1):
    RB = 1024
    grid = (N_PAD // RB,)

    def body(x_ref, w_ref, b_ref, d0_ref, d1_ref, hs_ref, n2_ref, invn_ref):
        h = jnp.dot(x_ref[...], w_ref[...],
                    preferred_element_type=jnp.float32) + b_ref[...]
        h = jnp.maximum(h, 0.0)
        dg = jnp.maximum(d0_ref[...] + d1_ref[...], 1.0)
        norm = lax.rsqrt(dg)
        hs_ref[...] = h * norm[:, None]
        n2_ref[...] = 1.0 / dg
        invn_ref[...] = jnp.sqrt(dg)

    return pl.pallas_call(
        body,
        grid=grid,
        in_specs=[
            pl.BlockSpec((RB, D_IN), lambda i: (i, 0)),
            pl.BlockSpec((D_IN, H), lambda i: (0, 0)),
            pl.BlockSpec((1, H), lambda i: (0, 0)),
            pl.BlockSpec((RB,), lambda i: (i,)),
            pl.BlockSpec((RB,), lambda i: (i,)),
        ],
        out_specs=[
            pl.BlockSpec((RB, H), lambda i: (i, 0)),
            pl.BlockSpec((RB,), lambda i: (i,)),
            pl.BlockSpec((RB,), lambda i: (i,)),
        ],
        out_shape=[
            jax.ShapeDtypeStruct((N_PAD, H), jnp.float32),
            jax.ShapeDtypeStruct((N_PAD,), jnp.float32),
            jax.ShapeDtypeStruct((N_PAD,), jnp.float32),
        ],
    )(x_pad, W_in, b_in2, d0, d1)


def _tc_proj(ts, invn, W6, b_rn2, gamma2, beta2):
    RB = 1024
    grid = (N_PAD // RB,)
    nt = len(ts)

    def body(*refs):
        t_refs = refs[:nt]
        invn_ref, w_ref, b_ref, g_ref, be_ref, z_ref = refs[nt:]
        invn = invn_ref[...][:, None]
        acc = jnp.zeros((RB, H), jnp.float32)
        for k in range(N_HOPS):
            hk = t_refs[k][...] * invn
            acc = acc + jnp.dot(hk, w_ref[k],
                                preferred_element_type=jnp.float32)
        z = acc + b_ref[...]
        mu = jnp.mean(z, axis=-1, keepdims=True)
        zc = z - mu
        var = jnp.mean(zc * zc, axis=-1, keepdims=True)
        z_ref[...] = zc * lax.rsqrt(var + 1e-5) * g_ref[...] + be_ref[...]

    in_specs = [pl.BlockSpec((RB, H), lambda i: (i, 0))] * nt + [
        pl.BlockSpec((RB,), lambda i: (i,)),
        pl.BlockSpec((N_HOPS, H, H), lambda i: (0, 0, 0)),
        pl.BlockSpec((1, H), lambda i: (0, 0)),
        pl.BlockSpec((1, H), lambda i: (0, 0)),
        pl.BlockSpec((1, H), lambda i: (0, 0)),
    ]
    return pl.pallas_call(
        body,
        grid=grid,
        in_specs=in_specs,
        out_specs=pl.BlockSpec((RB, H), lambda i: (i, 0)),
        out_shape=jax.ShapeDtypeStruct((N_PAD, H), jnp.float32),
    )(*ts, invn, W6, b_rn2, gamma2, beta2)


def kernel(x, edge_index, W_in, b_in, W_rn, b_rn, gamma, beta):
    pad = E_PAD - E
    pad_idx = jnp.arange(pad, dtype=jnp.int32)
    src2 = jnp.concatenate(
        [edge_index[0], pad_idx % N_PAD]).reshape(TOTB, BATCH)
    pad_dst = N + (pad_idx % (N_PAD - N))
    dst2 = jnp.concatenate([edge_index[1], pad_dst]).reshape(TOTB, BATCH)
    x_pad = jnp.pad(x, ((0, N_PAD - N), (0, 0)))

    d0, d1 = _sc_degree(dst2)
    hs0, n2, invn = _tc_prep(x_pad, W_in, b_in.reshape(1, H), d0, d1)
    ts = []
    table = hs0
    for _ in range(N_HOPS):
        p0, p1 = _sc_hop(table, src2, dst2)
        table = _tc_combine(p0, p1, n2)
        ts.append(table)
    z = _tc_proj(ts, invn, W_rn.reshape(N_HOPS, H, H),
                 b_rn.reshape(1, H), gamma.reshape(1, H), beta.reshape(1, H))
    return z[:N]


# speedup vs baseline: 4.4291x; 1.0755x over previous
import functools

import jax
import jax.numpy as jnp
from jax import lax
from jax.experimental import pallas as pl
from jax.experimental.pallas import tpu as pltpu
from jax.experimental.pallas import tpu_sc as plsc

N = 10000
E = 320000
D_IN = 128
H = 128
N_HOPS = 6

NC = 2
NS = 16
NW = NC * NS
BATCH = 64
NBW = 160
CHUNK = 32
NBUF = 4
DB = 128
TOTB = NW * NBW
E_PAD = TOTB * BATCH
N_PAD = 10240
RP = N_PAD // NS
RC = RP // BATCH


def _mesh():
    return plsc.VectorSubcoreMesh(
        core_axis_name="c", subcore_axis_name="s",
        num_cores=NC, num_subcores=NS)


def _sc_degree(dst2):
    @functools.partial(
        pl.kernel,
        out_type=(jax.ShapeDtypeStruct((N_PAD,), jnp.float32),
                  jax.ShapeDtypeStruct((N_PAD,), jnp.float32)),
        mesh=_mesh(),
        scratch_types=[
            pltpu.VMEM((NBW, BATCH), jnp.int32),
            pltpu.VMEM((BATCH,), jnp.float32),
            pltpu.VMEM((DB,), jnp.float32),
            pltpu.VMEM_SHARED((N_PAD,), jnp.float32),
        ],
    )
    def deg_kernel(dst_hbm, d0_out, d1_out, dst_v, ones_v, zb_v, deg_sh):
        c = lax.axis_index("c")
        s = lax.axis_index("s")
        w = s * NC + c
        for i in range(BATCH // 16):
            ones_v[pl.ds(i * 16, 16)] = jnp.ones((16,), jnp.float32)
        for i in range(DB // 16):
            zb_v[pl.ds(i * 16, 16)] = jnp.zeros((16,), jnp.float32)
        pltpu.sync_copy(dst_hbm.at[pl.ds(w * NBW, NBW)], dst_v)
        base = s * RP
        for t in range(RP // DB):
            pltpu.sync_copy(zb_v, deg_sh.at[pl.ds(base + t * DB, DB)])
        plsc.subcore_barrier()

        def body(j, carry):
            pltpu.sync_copy(ones_v, deg_sh.at[dst_v.at[j]], add=True)
            return carry

        lax.fori_loop(0, NBW, body, 0)
        plsc.subcore_barrier()

        @pl.when(c == 0)
        def _():
            for t in range(RP // DB):
                off = base + t * DB
                pltpu.sync_copy(deg_sh.at[pl.ds(off, DB)],
                                d0_out.at[pl.ds(off, DB)])

        @pl.when(c == 1)
        def _():
            for t in range(RP // DB):
                off = base + t * DB
                pltpu.sync_copy(deg_sh.at[pl.ds(off, DB)],
                                d1_out.at[pl.ds(off, DB)])

    return deg_kernel(dst2)


def _sc_hop(table, src2, dst2):
    @functools.partial(
        pl.kernel,
        out_type=(jax.ShapeDtypeStruct((N_PAD, H), jnp.float32),
                  jax.ShapeDtypeStruct((N_PAD, H), jnp.float32)),
        mesh=_mesh(),
        scratch_types=[
            pltpu.VMEM((CHUNK, BATCH), jnp.int32),
            pltpu.VMEM((CHUNK, BATCH), jnp.int32),
            pltpu.VMEM((NBUF, BATCH, H), jnp.float32),
            pltpu.SemaphoreType.DMA,
            pltpu.SemaphoreType.DMA,
            pltpu.SemaphoreType.DMA,
            pltpu.SemaphoreType.DMA,
            pltpu.VMEM_SHARED((N_PAD, H), jnp.float32),
        ],
    )
    def hop_kernel(table_hbm, src_hbm, dst_hbm, p0_out, p1_out,
                   src_v, dst_v, rows_v, gsem0, gsem1, gsem2, gsem3, agg_sh):
        c = lax.axis_index("c")
        s = lax.axis_index("s")
        w = s * NC + c
        base_b = w * NBW
        gsems = (gsem0, gsem1, gsem2, gsem3)

        def zrow(i, carry):
            for cc in range(H // 16):
                rows_v[0, i, pl.ds(cc * 16, 16)] = jnp.zeros(
                    (16,), jnp.float32)
            return carry

        lax.fori_loop(0, BATCH, zrow, 0)
        base = s * RP
        for t in range(RC):
            pltpu.sync_copy(rows_v.at[0],
                            agg_sh.at[pl.ds(base + t * BATCH, BATCH)])
        plsc.subcore_barrier()

        def chunk_body(g, carry):
            off = base_b + g * CHUNK
            pltpu.sync_copy(src_hbm.at[pl.ds(off, CHUNK)], src_v)
            pltpu.sync_copy(dst_hbm.at[pl.ds(off, CHUNK)], dst_v)
            for p in range(NBUF):
                pltpu.async_copy(table_hbm.at[src_v.at[p]], rows_v.at[p],
                                 gsems[p])

            def quad_body(q, carry2):
                for p in range(NBUF):
                    j = NBUF * q + p
                    pltpu.make_async_copy(
                        table_hbm.at[src_v.at[j]], rows_v.at[p],
                        gsems[p]).wait()
                    pltpu.sync_copy(rows_v.at[p], agg_sh.at[dst_v.at[j]],
                                    add=True)

                    @pl.when(q < CHUNK // NBUF - 1)
                    def _():
                        pltpu.async_copy(
                            table_hbm.at[src_v.at[j + NBUF]], rows_v.at[p],
                            gsems[p])
                return carry2

            lax.fori_loop(0, CHUNK // NBUF, quad_body, 0)
            return carry

        lax.fori_loop(0, NBW // CHUNK, chunk_body, 0)
        plsc.subcore_barrier()

        @pl.when(c == 0)
        def _():
            for t in range(RC):
                off = base + t * BATCH
                pltpu.sync_copy(agg_sh.at[pl.ds(off, BATCH)],
                                p0_out.at[pl.ds(off, BATCH)])

        @pl.when(c == 1)
        def _():
            for t in range(RC):
                off = base + t * BATCH
                pltpu.sync_copy(agg_sh.at[pl.ds(off, BATCH)],
                                p1_out.at[pl.ds(off, BATCH)])

    return hop_kernel(table, src2, dst2)


def _tc_combine(p0, p1, n2):
    RB = 1024
    grid = (N_PAD // RB,)

    def body(p0_ref, p1_ref, n2_ref, t_ref):
        t_ref[...] = (p0_ref[...] + p1_ref[...]) * n2_ref[...][:, None]

    return pl.pallas_call(
        body,
        grid=grid,
        in_specs=[
            pl.BlockSpec((RB, H), lambda i: (i, 0)),
            pl.BlockSpec((RB, H), lambda i: (i, 0)),
            pl.BlockSpec((RB,), lambda i: (i,)),
        ],
        out_specs=pl.BlockSpec((RB, H), lambda i: (i, 0)),
        out_shape=jax.ShapeDtypeStruct((N_PAD, H), jnp.float32),
    )(p0, p1, n2)


def _tc_prep(x_pad, W_in, b_in2, d0, d1):
    RB = 1024
    grid = (N_PAD // RB,)

    def body(x_ref, w_ref, b_ref, d0_ref, d1_ref, hs_ref, n2_ref, invn_ref):
        h = jnp.dot(x_ref[...], w_ref[...],
                    preferred_element_type=jnp.float32) + b_ref[...]
        h = jnp.maximum(h, 0.0)
        dg = jnp.maximum(d0_ref[...] + d1_ref[...], 1.0)
        norm = lax.rsqrt(dg)
        hs_ref[...] = h * norm[:, None]
        n2_ref[...] = 1.0 / dg
        invn_ref[...] = jnp.sqrt(dg)

    return pl.pallas_call(
        body,
        grid=grid,
        in_specs=[
            pl.BlockSpec((RB, D_IN), lambda i: (i, 0)),
            pl.BlockSpec((D_IN, H), lambda i: (0, 0)),
            pl.BlockSpec((1, H), lambda i: (0, 0)),
            pl.BlockSpec((RB,), lambda i: (i,)),
            pl.BlockSpec((RB,), lambda i: (i,)),
        ],
        out_specs=[
            pl.BlockSpec((RB, H), lambda i: (i, 0)),
            pl.BlockSpec((RB,), lambda i: (i,)),
            pl.BlockSpec((RB,), lambda i: (i,)),
        ],
        out_shape=[
            jax.ShapeDtypeStruct((N_PAD, H), jnp.float32),
            jax.ShapeDtypeStruct((N_PAD,), jnp.float32),
            jax.ShapeDtypeStruct((N_PAD,), jnp.float32),
        ],
    )(x_pad, W_in, b_in2, d0, d1)


def _tc_proj(ts, invn, W6, b_rn2, gamma2, beta2):
    RB = 1024
    grid = (N_PAD // RB,)
    nt = len(ts)

    def body(*refs):
        t_refs = refs[:nt]
        invn_ref, w_ref, b_ref, g_ref, be_ref, z_ref = refs[nt:]
        invn = invn_ref[...][:, None]
        acc = jnp.zeros((RB, H), jnp.float32)
        for k in range(N_HOPS):
            hk = t_refs[k][...] * invn
            acc = acc + jnp.dot(hk, w_ref[k],
                                preferred_element_type=jnp.float32)
        z = acc + b_ref[...]
        mu = jnp.mean(z, axis=-1, keepdims=True)
        zc = z - mu
        var = jnp.mean(zc * zc, axis=-1, keepdims=True)
        z_ref[...] = zc * lax.rsqrt(var + 1e-5) * g_ref[...] + be_ref[...]

    in_specs = [pl.BlockSpec((RB, H), lambda i: (i, 0))] * nt + [
        pl.BlockSpec((RB,), lambda i: (i,)),
        pl.BlockSpec((N_HOPS, H, H), lambda i: (0, 0, 0)),
        pl.BlockSpec((1, H), lambda i: (0, 0)),
        pl.BlockSpec((1, H), lambda i: (0, 0)),
        pl.BlockSpec((1, H), lambda i: (0, 0)),
    ]
    return pl.pallas_call(
        body,
        grid=grid,
        in_specs=in_specs,
        out_specs=pl.BlockSpec((RB, H), lambda i: (i, 0)),
        out_shape=jax.ShapeDtypeStruct((N_PAD, H), jnp.float32),
    )(*ts, invn, W6, b_rn2, gamma2, beta2)


def kernel(x, edge_index, W_in, b_in, W_rn, b_rn, gamma, beta):
    pad = E_PAD - E
    pad_idx = jnp.arange(pad, dtype=jnp.int32)
    src2 = jnp.concatenate(
        [edge_index[0], pad_idx % N_PAD]).reshape(TOTB, BATCH)
    pad_dst = N + (pad_idx % (N_PAD - N))
    dst2 = jnp.concatenate([edge_index[1], pad_dst]).reshape(TOTB, BATCH)
    x_pad = jnp.pad(x, ((0, N_PAD - N), (0, 0)))

    d0, d1 = _sc_degree(dst2)
    hs0, n2, invn = _tc_prep(x_pad, W_in, b_in.reshape(1, H), d0, d1)
    ts = []
    table = hs0
    for _ in range(N_HOPS):
        p0, p1 = _sc_hop(table, src2, dst2)
        table = _tc_combine(p0, p1, n2)
        ts.append(table)
    z = _tc_proj(ts, invn, W_rn.reshape(N_HOPS, H, H),
                 b_rn.reshape(1, H), gamma.reshape(1, H), beta.reshape(1, H))
    return z[:N]
